# Initial kernel scaffold; baseline (speedup 1.0000x reference)
import jax
import jax.numpy as jnp
from jax.experimental import pallas as pl


def kernel(x, expert_weights, top_k_indices, W, b):
    raise NotImplementedError("write your pallas kernel here")


# speedup vs baseline: 5.2620x; 5.2620x over previous
import functools

import jax
import jax.numpy as jnp
from jax.experimental import pallas as pl


def _moe_dense_body(x_ref, ew_ref, idx_ref, w_ref, b_ref, o_ref, *, n_experts):
    xb = x_ref[...]
    ew = ew_ref[...]
    idx = idx_ref[...]
    acc = jnp.zeros(o_ref.shape, jnp.float32)
    for e in range(n_experts):
        coef = jnp.sum(ew * (idx == e).astype(jnp.float32), axis=1, keepdims=True)
        mm = jax.lax.dot_general(
            xb, w_ref[e],
            (((1,), (1,)), ((), ())),
            preferred_element_type=jnp.float32,
        )
        acc = acc + coef * (mm + b_ref[e][None, :])
    o_ref[...] = acc


def kernel(x, expert_weights, top_k_indices, W, b):
    B, S, D = x.shape
    E, O, _ = W.shape
    K = expert_weights.shape[-1]
    T = B * S

    x2 = x.reshape(T, D).astype(jnp.bfloat16)
    Wb = W.astype(jnp.bfloat16)
    ew = expert_weights.reshape(T, K)
    idx = top_k_indices.reshape(T, K).astype(jnp.int32)

    TM = min(512, T)
    ON = min(512, O)
    grid = (O // ON, T // TM)

    out = pl.pallas_call(
        functools.partial(_moe_dense_body, n_experts=E),
        grid=grid,
        in_specs=[
            pl.BlockSpec((TM, D), lambda j, i: (i, 0)),
            pl.BlockSpec((TM, K), lambda j, i: (i, 0)),
            pl.BlockSpec((TM, K), lambda j, i: (i, 0)),
            pl.BlockSpec((E, ON, D), lambda j, i: (0, j, 0)),
            pl.BlockSpec((E, ON), lambda j, i: (0, j)),
        ],
        out_specs=pl.BlockSpec((TM, ON), lambda j, i: (i, j)),
        out_shape=jax.ShapeDtypeStruct((T, O), jnp.float32),
    )(x2, ew, idx, Wb, b)
    return out.reshape(B, S, O)
